# Initial kernel scaffold; baseline (speedup 1.0000x reference)
import functools

import jax
import jax.numpy as jnp
from jax import lax
from jax.experimental import pallas as pl
from jax.experimental.pallas import tpu as pltpu
from jax.experimental.pallas import tpu_sc as plsc

N_NODES = 10000
D_FEAT = 128
NUM_CLUSTERS = 5000

NC = 2
NS = 16
NW = NC * NS

PER_W = 320
CHUNK = 80
NCHUNK = PER_W // CHUNK
NODES_PAD = NW * PER_W

SEG_PER_TILE = 320
SEG_PAD = NS * SEG_PER_TILE
CNT_W = 16

_mesh = plsc.VectorSubcoreMesh(core_axis_name="c", subcore_axis_name="s")


@functools.partial(
    pl.kernel,
    out_type=(
        jax.ShapeDtypeStruct((NC, SEG_PAD, D_FEAT), jnp.float32),
        jax.ShapeDtypeStruct((NC, SEG_PAD, CNT_W), jnp.float32),
    ),
    mesh=_mesh,
    scratch_types=[
        pltpu.VMEM((NCHUNK, CHUNK, D_FEAT), jnp.float32),
        pltpu.VMEM((NCHUNK, CHUNK), jnp.int32),
        pltpu.VMEM((CHUNK, CNT_W), jnp.float32),
        pltpu.VMEM((SEG_PER_TILE, D_FEAT), jnp.float32),
        pltpu.VMEM((SEG_PER_TILE, CNT_W), jnp.float32),
        pltpu.VMEM_SHARED((SEG_PAD, D_FEAT), jnp.float32),
        pltpu.VMEM_SHARED((SEG_PAD, CNT_W), jnp.float32),
        pltpu.SemaphoreType.DMA,
        pltpu.SemaphoreType.DMA,
    ],
)
def _sc_accumulate(x_hbm, cid_hbm, sums_out, cnts_out,
                   rows_v, cid_v, ones_v, z_v, zc_v, sums_sh, cnts_sh,
                   sem_x, sem_c):
    c = lax.axis_index("c")
    s = lax.axis_index("s")
    wid = c * NS + s

    cp_x = pltpu.make_async_copy(x_hbm.at[wid], rows_v, sem_x)
    cp_c = pltpu.make_async_copy(cid_hbm.at[wid], cid_v, sem_c)
    cp_x.start()
    cp_c.start()

    zeros16 = jnp.zeros((16,), jnp.float32)
    ones16 = jnp.ones((16,), jnp.float32)

    def fill_zeros(i, carry):
        for k in range(D_FEAT // 16):
            z_v[i, pl.ds(k * 16, 16)] = zeros16
        zc_v[i, :] = zeros16
        return carry

    lax.fori_loop(0, SEG_PER_TILE, fill_zeros, 0)

    def fill_ones(i, carry):
        ones_v[i, :] = ones16
        return carry

    lax.fori_loop(0, CHUNK, fill_ones, 0)

    seg0 = s * SEG_PER_TILE
    pltpu.sync_copy(z_v, sums_sh.at[pl.ds(seg0, SEG_PER_TILE)])
    pltpu.sync_copy(zc_v, cnts_sh.at[pl.ds(seg0, SEG_PER_TILE)])

    cp_x.wait()
    cp_c.wait()
    plsc.subcore_barrier()

    for j in range(NCHUNK):
        pltpu.sync_copy(rows_v.at[j], sums_sh.at[cid_v.at[j]], add=True)
        pltpu.sync_copy(ones_v, cnts_sh.at[cid_v.at[j]], add=True)

    plsc.subcore_barrier()

    pltpu.sync_copy(sums_sh.at[pl.ds(seg0, SEG_PER_TILE)],
                    sums_out.at[c, pl.ds(seg0, SEG_PER_TILE)])
    pltpu.sync_copy(cnts_sh.at[pl.ds(seg0, SEG_PER_TILE)],
                    cnts_out.at[c, pl.ds(seg0, SEG_PER_TILE)])


def _combine_body(s_ref, c_ref, o_ref):
    sm = s_ref[0] + s_ref[1]
    cnt = c_ref[0, :, 0:1] + c_ref[1, :, 0:1]
    o_ref[...] = sm / jnp.maximum(cnt, 1.0)


_BLK = 640


def _combine(sums_p, cnts_p):
    return pl.pallas_call(
        _combine_body,
        grid=(SEG_PAD // _BLK,),
        in_specs=[
            pl.BlockSpec((NC, _BLK, D_FEAT), lambda i: (0, i, 0)),
            pl.BlockSpec((NC, _BLK, CNT_W), lambda i: (0, i, 0)),
        ],
        out_specs=pl.BlockSpec((_BLK, D_FEAT), lambda i: (i, 0)),
        out_shape=jax.ShapeDtypeStruct((SEG_PAD, D_FEAT), jnp.float32),
    )(sums_p, cnts_p)


def kernel(x, edge_index, cluster):
    del edge_index
    x_pad = jnp.zeros((NODES_PAD, D_FEAT), jnp.float32).at[:N_NODES].set(x)
    cid_pad = jnp.full((NODES_PAD,), NUM_CLUSTERS, jnp.int32).at[:N_NODES].set(cluster)
    x_r = x_pad.reshape(NW, NCHUNK, CHUNK, D_FEAT)
    cid_r = cid_pad.reshape(NW, NCHUNK, CHUNK)
    sums_p, cnts_p = _sc_accumulate(x_r, cid_r)
    return _combine(sums_p, cnts_p)[:NUM_CLUSTERS]


# speedup vs baseline: 3.3182x; 3.3182x over previous
import functools

import jax
import jax.numpy as jnp
from jax import lax
from jax.experimental import pallas as pl
from jax.experimental.pallas import tpu as pltpu
from jax.experimental.pallas import tpu_sc as plsc

N_NODES = 10000
D_FEAT = 128
NUM_CLUSTERS = 5000

NC = 2
NS = 16
NW = NC * NS

PER_W = 320
CHUNK = 80
NCHUNK = PER_W // CHUNK
NODES_PAD = NW * PER_W

SEG_PER_TILE = 320
SEG_PAD = NS * SEG_PER_TILE
Z_ROWS = 64

_mesh = plsc.VectorSubcoreMesh(core_axis_name="c", subcore_axis_name="s")


@functools.partial(
    pl.kernel,
    out_type=(
        jax.ShapeDtypeStruct((NC * SEG_PAD, D_FEAT), jnp.float32),
        jax.ShapeDtypeStruct((NC * SEG_PAD, D_FEAT), jnp.float32),
    ),
    mesh=_mesh,
    scratch_types=[
        pltpu.VMEM((2, CHUNK, D_FEAT), jnp.float32),
        pltpu.VMEM((NCHUNK, CHUNK), jnp.int32),
        pltpu.VMEM((CHUNK, D_FEAT), jnp.float32),
        pltpu.VMEM((Z_ROWS, D_FEAT), jnp.float32),
        pltpu.VMEM_SHARED((SEG_PAD, D_FEAT), jnp.float32),
        pltpu.VMEM_SHARED((SEG_PAD, D_FEAT), jnp.float32),
        pltpu.SemaphoreType.DMA,
        pltpu.SemaphoreType.DMA,
    ],
)
def _sc_accumulate(x_hbm, cid_hbm, sums_out, cnts_out,
                   rows_v, cid_v, ones_v, z_v, sums_sh, cnts_sh, sem0, sem1):
    c = lax.axis_index("c")
    s = lax.axis_index("s")
    wid = c * NS + s
    base = wid * PER_W

    sems = [sem0, sem1]
    copies = []
    for j in range(NCHUNK):
        copies.append(pltpu.make_async_copy(
            x_hbm.at[pl.ds(base + j * CHUNK, CHUNK)], rows_v.at[j % 2], sems[j % 2]))
    copies[0].start()
    pltpu.sync_copy(cid_hbm.at[pl.ds(wid * NCHUNK, NCHUNK)], cid_v)

    zeros16 = jnp.zeros((16,), jnp.float32)
    ones16 = jnp.ones((16,), jnp.float32)
    for i in range(Z_ROWS):
        for k in range(D_FEAT // 16):
            z_v[i, pl.ds(k * 16, 16)] = zeros16
    for i in range(CHUNK):
        for k in range(D_FEAT // 16):
            ones_v[i, pl.ds(k * 16, 16)] = ones16

    seg0 = s * SEG_PER_TILE
    for i in range(SEG_PER_TILE // Z_ROWS):
        pltpu.sync_copy(z_v, sums_sh.at[pl.ds(seg0 + i * Z_ROWS, Z_ROWS)])
        pltpu.sync_copy(z_v, cnts_sh.at[pl.ds(seg0 + i * Z_ROWS, Z_ROWS)])

    plsc.subcore_barrier()

    for j in range(NCHUNK):
        copies[j].wait()
        if j + 1 < NCHUNK:
            copies[j + 1].start()
        pltpu.sync_copy(rows_v.at[j % 2], sums_sh.at[cid_v.at[j]], add=True)
        pltpu.sync_copy(ones_v, cnts_sh.at[cid_v.at[j]], add=True)

    plsc.subcore_barrier()

    out0 = c * SEG_PAD + seg0
    pltpu.sync_copy(sums_sh.at[pl.ds(seg0, SEG_PER_TILE)],
                    sums_out.at[pl.ds(out0, SEG_PER_TILE)])
    pltpu.sync_copy(cnts_sh.at[pl.ds(seg0, SEG_PER_TILE)],
                    cnts_out.at[pl.ds(out0, SEG_PER_TILE)])


def _combine_body(s_ref, c_ref, o_ref):
    sm = s_ref[0] + s_ref[1]
    cnt = c_ref[0, :, 0:1] + c_ref[1, :, 0:1]
    o_ref[...] = sm / jnp.maximum(cnt, 1.0)


_BLK = 640


def _combine(sums_p, cnts_p):
    return pl.pallas_call(
        _combine_body,
        grid=(SEG_PAD // _BLK,),
        in_specs=[
            pl.BlockSpec((NC, _BLK, D_FEAT), lambda i: (0, i, 0)),
            pl.BlockSpec((NC, _BLK, D_FEAT), lambda i: (0, i, 0)),
        ],
        out_specs=pl.BlockSpec((_BLK, D_FEAT), lambda i: (i, 0)),
        out_shape=jax.ShapeDtypeStruct((SEG_PAD, D_FEAT), jnp.float32),
    )(sums_p, cnts_p)


def kernel(x, edge_index, cluster):
    del edge_index
    x_pad = jnp.zeros((NODES_PAD, D_FEAT), jnp.float32).at[:N_NODES].set(x)
    cid_pad = jnp.full((NODES_PAD,), NUM_CLUSTERS, jnp.int32).at[:N_NODES].set(cluster)
    cid_r = cid_pad.reshape(NW * NCHUNK, CHUNK)
    sums_f, cnts_f = _sc_accumulate(x_pad, cid_r)
    sums_p = sums_f.reshape(NC, SEG_PAD, D_FEAT)
    cnts_p = cnts_f.reshape(NC, SEG_PAD, D_FEAT)
    return _combine(sums_p, cnts_p)[:NUM_CLUSTERS]


# speedup vs baseline: 3.3454x; 1.0082x over previous
import functools

import jax
import jax.numpy as jnp
from jax import lax
from jax.experimental import pallas as pl
from jax.experimental.pallas import tpu as pltpu
from jax.experimental.pallas import tpu_sc as plsc

N_NODES = 10000
D_FEAT = 128
NUM_CLUSTERS = 5000

NC = 2
NS = 16

NODES_PAD = 10240
NPT = NODES_PAD // NS
CHUNK = 80
NCHUNK = NPT // CHUNK

SEGS_CORE = 2560
SEG_PAD_CORE = 2688
ZERO_PER_TILE = SEG_PAD_CORE // NS
Z_ROWS = 84
DIV_PER_TILE = SEGS_CORE // NS
DUMP = SEGS_CORE

_mesh = plsc.VectorSubcoreMesh(core_axis_name="c", subcore_axis_name="s")


@functools.partial(
    pl.kernel,
    out_type=jax.ShapeDtypeStruct((NC * SEGS_CORE, D_FEAT), jnp.float32),
    mesh=_mesh,
    scratch_types=[
        pltpu.VMEM((2, CHUNK, D_FEAT), jnp.float32),
        pltpu.VMEM((NCHUNK, CHUNK), jnp.int32),
        pltpu.VMEM((NCHUNK, CHUNK), jnp.int32),
        pltpu.VMEM((CHUNK, D_FEAT), jnp.float32),
        pltpu.VMEM((Z_ROWS, D_FEAT), jnp.float32),
        pltpu.VMEM((CHUNK, D_FEAT), jnp.float32),
        pltpu.VMEM((CHUNK, D_FEAT), jnp.float32),
        pltpu.VMEM_SHARED((SEG_PAD_CORE, D_FEAT), jnp.float32),
        pltpu.VMEM_SHARED((SEG_PAD_CORE, D_FEAT), jnp.float32),
        pltpu.SemaphoreType.DMA,
        pltpu.SemaphoreType.DMA,
    ],
)
def _sc_pool(x_hbm, cid_hbm, out_hbm,
             rows_v, cid_v, cid2_v, ones_v, z_v, sv, cv,
             sums_sh, cnts_sh, sem0, sem1):
    c = lax.axis_index("c")
    s = lax.axis_index("s")
    base = s * NPT

    sems = [sem0, sem1]
    copies = [
        pltpu.make_async_copy(
            x_hbm.at[pl.ds(base + j * CHUNK, CHUNK)], rows_v.at[j % 2], sems[j % 2])
        for j in range(NCHUNK)
    ]
    copies[0].start()
    pltpu.sync_copy(cid_hbm.at[pl.ds(s * NCHUNK, NCHUNK)], cid_v)

    lo = c * SEGS_CORE
    dump16 = jnp.full((16,), DUMP, jnp.int32)
    for r in range(NCHUNK):
        for k in range(CHUNK // 16):
            ids = cid_v[r, pl.ds(k * 16, 16)]
            loc = ids - lo
            ok = (loc >= 0) & (loc < SEGS_CORE)
            cid2_v[r, pl.ds(k * 16, 16)] = jnp.where(ok, loc, dump16)

    zeros16 = jnp.zeros((16,), jnp.float32)
    ones16 = jnp.ones((16,), jnp.float32)
    for i in range(Z_ROWS):
        for k in range(D_FEAT // 16):
            z_v[i, pl.ds(k * 16, 16)] = zeros16
    for i in range(CHUNK):
        for k in range(D_FEAT // 16):
            ones_v[i, pl.ds(k * 16, 16)] = ones16

    zb = s * ZERO_PER_TILE
    for i in range(ZERO_PER_TILE // Z_ROWS):
        pltpu.sync_copy(z_v, sums_sh.at[pl.ds(zb + i * Z_ROWS, Z_ROWS)])
        pltpu.sync_copy(z_v, cnts_sh.at[pl.ds(zb + i * Z_ROWS, Z_ROWS)])

    plsc.subcore_barrier()

    for j in range(NCHUNK):
        copies[j].wait()
        if j + 1 < NCHUNK:
            copies[j + 1].start()
        pltpu.sync_copy(rows_v.at[j % 2], sums_sh.at[cid2_v.at[j]], add=True)
        pltpu.sync_copy(ones_v, cnts_sh.at[cid2_v.at[j]], add=True)

    plsc.subcore_barrier()

    for rnd in range(DIV_PER_TILE // CHUNK):
        r0 = s * DIV_PER_TILE + rnd * CHUNK
        pltpu.sync_copy(sums_sh.at[pl.ds(r0, CHUNK)], sv)
        pltpu.sync_copy(cnts_sh.at[pl.ds(r0, CHUNK)], cv)
        for j in range(CHUNK):
            cnt = cv[j, pl.ds(0, 16)]
            rec = ones16 / jnp.maximum(cnt, ones16)
            for k in range(D_FEAT // 16):
                sv[j, pl.ds(k * 16, 16)] = sv[j, pl.ds(k * 16, 16)] * rec
        pltpu.sync_copy(sv, out_hbm.at[pl.ds(c * SEGS_CORE + r0, CHUNK)])


def kernel(x, edge_index, cluster):
    del edge_index
    x_pad = jnp.zeros((NODES_PAD, D_FEAT), jnp.float32).at[:N_NODES].set(x)
    cid_pad = jnp.full((NODES_PAD,), NUM_CLUSTERS, jnp.int32).at[:N_NODES].set(cluster)
    cid_r = cid_pad.reshape(NS * NCHUNK, CHUNK)
    return _sc_pool(x_pad, cid_r)[:NUM_CLUSTERS]


# speedup vs baseline: 3.4221x; 1.0229x over previous
import functools

import jax
import jax.numpy as jnp
from jax import lax
from jax.experimental import pallas as pl
from jax.experimental.pallas import tpu as pltpu
from jax.experimental.pallas import tpu_sc as plsc

N_NODES = 10000
D_FEAT = 128
NUM_CLUSTERS = 5000

NC = 2
NS = 16

NODES_PAD = 10240
NPT = NODES_PAD // NS
CHUNK = 80
NCHUNK = NPT // CHUNK
XCLAMP = N_NODES - CHUNK

SEGS_CORE = 2560
SEG_PAD_CORE = 2688
DIV_PER_TILE = SEGS_CORE // NS
DUMP = SEGS_CORE

_mesh = plsc.VectorSubcoreMesh(core_axis_name="c", subcore_axis_name="s")


@functools.partial(
    pl.kernel,
    out_type=jax.ShapeDtypeStruct((NC * SEGS_CORE, D_FEAT), jnp.float32),
    mesh=_mesh,
    scratch_types=[
        pltpu.VMEM((2, CHUNK, D_FEAT), jnp.float32),
        pltpu.VMEM((NCHUNK, CHUNK), jnp.int32),
        pltpu.VMEM((NCHUNK, CHUNK), jnp.int32),
        pltpu.VMEM((CHUNK, D_FEAT), jnp.float32),
        pltpu.VMEM((CHUNK, D_FEAT), jnp.float32),
        pltpu.VMEM((CHUNK, D_FEAT), jnp.float32),
        pltpu.VMEM((CHUNK, D_FEAT), jnp.float32),
        pltpu.VMEM_SHARED((SEG_PAD_CORE, D_FEAT), jnp.float32),
        pltpu.VMEM_SHARED((SEG_PAD_CORE, D_FEAT), jnp.float32),
        pltpu.SemaphoreType.DMA,
        pltpu.SemaphoreType.DMA,
    ],
)
def _sc_pool(x_hbm, cid_hbm, out_hbm,
             rows_v, cid_v, cid2_v, ones_v, z_v, sv, cv,
             sums_sh, cnts_sh, sem0, sem1):
    c = lax.axis_index("c")
    s = lax.axis_index("s")
    base = s * NPT

    sems = [sem0, sem1]
    copies = [
        pltpu.make_async_copy(
            x_hbm.at[pl.ds(jnp.minimum(base + j * CHUNK, XCLAMP), CHUNK)],
            rows_v.at[j % 2], sems[j % 2])
        for j in range(NCHUNK)
    ]
    copies[0].start()
    pltpu.sync_copy(cid_hbm.at[pl.ds(s * NCHUNK, NCHUNK)], cid_v)

    lo = c * SEGS_CORE
    dump16 = jnp.full((16,), DUMP, jnp.int32)
    for r in range(NCHUNK):
        for k in range(CHUNK // 16):
            ids = cid_v[r, pl.ds(k * 16, 16)]
            loc = ids - lo
            ok = (loc >= 0) & (loc < SEGS_CORE)
            cid2_v[r, pl.ds(k * 16, 16)] = jnp.where(ok, loc, dump16)

    zeros16 = jnp.zeros((16,), jnp.float32)
    ones16 = jnp.ones((16,), jnp.float32)
    for i in range(CHUNK):
        for k in range(D_FEAT // 16):
            z_v[i, pl.ds(k * 16, 16)] = zeros16
            ones_v[i, pl.ds(k * 16, 16)] = ones16

    zb = s * DIV_PER_TILE
    for i in range(DIV_PER_TILE // CHUNK):
        pltpu.sync_copy(z_v, sums_sh.at[pl.ds(zb + i * CHUNK, CHUNK)])
        pltpu.sync_copy(z_v, cnts_sh.at[pl.ds(zb + i * CHUNK, CHUNK)])

    plsc.subcore_barrier()

    for j in range(NCHUNK):
        copies[j].wait()
        if j + 1 < NCHUNK:
            copies[j + 1].start()
        pltpu.sync_copy(rows_v.at[j % 2], sums_sh.at[cid2_v.at[j]], add=True)
        pltpu.sync_copy(ones_v, cnts_sh.at[cid2_v.at[j]], add=True)

    plsc.subcore_barrier()

    for rnd in range(DIV_PER_TILE // CHUNK):
        r0 = zb + rnd * CHUNK
        pltpu.sync_copy(sums_sh.at[pl.ds(r0, CHUNK)], sv)
        pltpu.sync_copy(cnts_sh.at[pl.ds(r0, CHUNK)], cv)
        for j in range(CHUNK):
            cnt = cv[j, pl.ds(0, 16)]
            rec = ones16 / jnp.maximum(cnt, ones16)
            for k in range(D_FEAT // 16):
                sv[j, pl.ds(k * 16, 16)] = sv[j, pl.ds(k * 16, 16)] * rec
        pltpu.sync_copy(sv, out_hbm.at[pl.ds(c * SEGS_CORE + r0, CHUNK)])


def kernel(x, edge_index, cluster):
    del edge_index
    cid_pad = jnp.full((NODES_PAD,), NUM_CLUSTERS, jnp.int32).at[:N_NODES].set(cluster)
    cid_r = cid_pad.reshape(NS * NCHUNK, CHUNK)
    return _sc_pool(x, cid_r)[:NUM_CLUSTERS]
